# baseline (speedup 1.0000x reference)
import functools

import jax
import jax.numpy as jnp
from jax import lax
from jax.experimental import pallas as pl
from jax.experimental.pallas import tpu as pltpu
from jax.experimental.pallas import tpu_sc as plsc

_NCHUNK = 2
_SLOTS = 2
_SUB = 32
_NBUF = 2


def _tc_copy(pos_embedding, nb):
    S, D = pos_embedding.shape
    CS = S // _NCHUNK

    def body(table_hbm, out_hbm, buf, insem, outsem):
        def in_copy(j):
            return pltpu.make_async_copy(
                table_hbm.at[pl.ds(j * CS, CS), :],
                buf.at[j % _SLOTS],
                insem.at[j % _SLOTS],
            )

        def out_copies(j):
            return [
                pltpu.make_async_copy(
                    buf.at[j % _SLOTS],
                    out_hbm.at[b, pl.ds(j * CS, CS), :],
                    outsem.at[j % _SLOTS],
                )
                for b in range(nb)
            ]

        pending = {}
        in_copy(0).start()
        for j in range(_NCHUNK):
            nxt = j + 1
            if nxt < _NCHUNK:
                prev = nxt - _SLOTS
                if prev >= 0:
                    for c in pending.pop(prev):
                        c.wait()
                in_copy(nxt).start()
            in_copy(j).wait()
            cs = out_copies(j)
            for c in cs:
                c.start()
            pending[j] = cs
        for j in sorted(pending):
            for c in pending[j]:
                c.wait()

    return pl.pallas_call(
        body,
        in_specs=[pl.BlockSpec(memory_space=pl.ANY)],
        out_specs=pl.BlockSpec(memory_space=pl.ANY),
        out_shape=jax.ShapeDtypeStruct((nb, S, D), pos_embedding.dtype),
        scratch_shapes=[
            pltpu.VMEM((_SLOTS, CS, D), pos_embedding.dtype),
            pltpu.SemaphoreType.DMA((_SLOTS,)),
            pltpu.SemaphoreType.DMA((_SLOTS,)),
        ],
    )(pos_embedding)


def _sc_copy(pos_embedding, nb):
    S, D = pos_embedding.shape
    info = plsc.get_sparse_core_info()
    NC, NS = info.num_cores, info.num_subcores
    NW = NC * NS
    RPW = S // NW
    NSUB = RPW // _SUB

    mesh = plsc.VectorSubcoreMesh(core_axis_name="c", subcore_axis_name="s")

    @functools.partial(
        pl.kernel,
        mesh=mesh,
        out_type=jax.ShapeDtypeStruct((nb, S, D), pos_embedding.dtype),
        scratch_types=[
            pltpu.VMEM((_NBUF, _SUB, D), pos_embedding.dtype),
            pltpu.SemaphoreType.DMA((_NBUF,)),
            pltpu.SemaphoreType.DMA((_NBUF,)),
        ],
    )
    def body(table_hbm, out_hbm, buf, insem, outsem):
        wid = lax.axis_index("s") * NC + lax.axis_index("c")
        base = wid * RPW

        def in_copy(j):
            return pltpu.make_async_copy(
                table_hbm.at[pl.ds(base + j * _SUB, _SUB), :],
                buf.at[j % _NBUF],
                insem.at[j % _NBUF],
            )

        def out_copies(j):
            return [
                pltpu.make_async_copy(
                    buf.at[j % _NBUF],
                    out_hbm.at[b, pl.ds(base + j * _SUB, _SUB), :],
                    outsem.at[j % _NBUF],
                )
                for b in range(nb)
            ]

        pending = {}
        in_copy(0).start()
        for j in range(NSUB):
            nxt = j + 1
            if nxt < NSUB:
                prev = nxt - _NBUF
                if prev >= 0:
                    for c in pending.pop(prev):
                        c.wait()
                in_copy(nxt).start()
            in_copy(j).wait()
            cs = out_copies(j)
            for c in cs:
                c.start()
            pending[j] = cs
        for j in sorted(pending):
            for c in pending[j]:
                c.wait()

    return body(pos_embedding)


def kernel(inputs, pos_embedding):
    B, S = inputs.shape
    tc_part = _tc_copy(pos_embedding, B - 1)
    sc_part = _sc_copy(pos_embedding, 1)
    return jnp.concatenate([tc_part, sc_part], axis=0)


# speedup vs baseline: 2.1847x; 2.1847x over previous
import functools

import jax
import jax.numpy as jnp
from jax import lax
from jax.experimental import pallas as pl
from jax.experimental.pallas import tpu as pltpu
from jax.experimental.pallas import tpu_sc as plsc

_SUB = 32
_NBUF = 3


def kernel(inputs, pos_embedding):
    B, S = inputs.shape
    P, D = pos_embedding.shape

    info = plsc.get_sparse_core_info()
    NC, NS = info.num_cores, info.num_subcores
    NW = NC * NS
    RPW = S // NW
    NSUB = RPW // _SUB

    mesh = plsc.VectorSubcoreMesh(core_axis_name="c", subcore_axis_name="s")

    @functools.partial(
        pl.kernel,
        mesh=mesh,
        out_type=jax.ShapeDtypeStruct((B, S, D), pos_embedding.dtype),
        scratch_types=[
            pltpu.VMEM((_NBUF, _SUB, D), pos_embedding.dtype),
            pltpu.SemaphoreType.DMA((_NBUF,)),
            pltpu.SemaphoreType.DMA((_NBUF,)),
        ],
    )
    def sc_copy(table_hbm, out_hbm, buf, insem, outsem):
        wid = lax.axis_index("s") * NC + lax.axis_index("c")
        base = wid * RPW

        def in_copy(j):
            return pltpu.make_async_copy(
                table_hbm.at[pl.ds(base + j * _SUB, _SUB), :],
                buf.at[j % _NBUF],
                insem.at[j % _NBUF],
            )

        def out_copies(j):
            return [
                pltpu.make_async_copy(
                    buf.at[j % _NBUF],
                    out_hbm.at[b, pl.ds(base + j * _SUB, _SUB), :],
                    outsem.at[j % _NBUF],
                )
                for b in range(B)
            ]

        pending = {}
        in_copy(0).start()
        for j in range(NSUB):
            nxt = j + 1
            if nxt < NSUB:
                prev = nxt - _NBUF
                if prev >= 0:
                    for c in pending.pop(prev):
                        c.wait()
                in_copy(nxt).start()
            in_copy(j).wait()
            cs = out_copies(j)
            for c in cs:
                c.start()
            pending[j] = cs
        for j in sorted(pending):
            for c in pending[j]:
                c.wait()

    return sc_copy(pos_embedding)
